# Initial kernel scaffold; baseline (speedup 1.0000x reference)
import functools

import jax
import jax.numpy as jnp
from jax import lax
from jax.experimental import pallas as pl
from jax.experimental.pallas import tpu as pltpu
from jax.experimental.pallas import tpu_sc as plsc

NUM_FIELDS = 26
FIELD_SIZE = 40000
CONT = 13
BATCH = 16384

NC = 2
NS = 16
L = 16
NW = NC * NS
BPW = BATCH // NW
NGB = BPW // 128
NROWS_IDX = NUM_FIELDS * NGB


def _fm_body(x_hbm, xc_hbm, tab_hbm, w_hbm, b_hbm, out_hbm,
             x_v, xc_v, w_v, b_v, idx_v, val_v, out_v, sem):
    cid = lax.axis_index("c")
    sid = lax.axis_index("s")
    wid = sid * NC + cid
    base = wid * BPW

    pltpu.sync_copy(x_hbm.at[pl.ds(base, BPW), :], x_v)
    pltpu.sync_copy(xc_hbm.at[pl.ds(base, BPW), :], xc_v)
    pltpu.sync_copy(w_hbm, w_v)
    pltpu.sync_copy(b_hbm, b_v)

    lanes = lax.iota(jnp.int32, 16)

    for g in range(NGB):
        def build(cc, carry, g=g):
            samp = (g * 8 + cc) * L + lanes
            lane0 = cc * L
            for j in range(NUM_FIELDS):
                col = jnp.full((L,), j, jnp.int32)
                xv = plsc.load_gather(x_v, [samp, col])
                idx_v[NGB * j + g, pl.ds(lane0, L)] = xv + (FIELD_SIZE * j)
            return carry
        lax.fori_loop(0, 8, build, 0)

    W = 16

    def fire(r, carry):
        pltpu.async_copy(tab_hbm.at[idx_v.at[r]], val_v.at[r], sem)
        return carry

    def drain_fire(r, carry):
        pltpu.make_async_copy(tab_hbm.at[idx_v.at[0]], val_v.at[0], sem).wait()
        pltpu.async_copy(tab_hbm.at[idx_v.at[r + W]], val_v.at[r + W], sem)
        return carry

    def drain(r, carry):
        pltpu.make_async_copy(tab_hbm.at[idx_v.at[0]], val_v.at[0], sem).wait()
        return carry

    lax.fori_loop(0, W, fire, 0)
    lax.fori_loop(0, NROWS_IDX - W, drain_fire, 0)
    lax.fori_loop(0, W, drain, 0)

    wjs = [plsc.load_gather(w_v, [jnp.full((L,), j, jnp.int32)])
           for j in range(CONT)]
    bias_vec = b_v[:]
    for g in range(NGB):
        def accum(cc, carry, g=g):
            lane0 = cc * L
            acc = bias_vec
            for j in range(NUM_FIELDS):
                acc = acc + val_v[NGB * j + g, pl.ds(lane0, L)]
            samp = (g * 8 + cc) * L + lanes
            for j in range(CONT):
                col = jnp.full((L,), j, jnp.int32)
                acc = acc + plsc.load_gather(xc_v, [samp, col]) * wjs[j]
            out_v[pl.ds(g * 128 + lane0, L)] = acc
            return carry
        lax.fori_loop(0, 8, accum, 0)

    pltpu.sync_copy(out_v, out_hbm.at[pl.ds(base, BPW)])


@functools.partial(
    pl.kernel,
    mesh=plsc.VectorSubcoreMesh(core_axis_name="c", subcore_axis_name="s"),
    out_type=jax.ShapeDtypeStruct((BATCH,), jnp.float32),
    scratch_types=[
        pltpu.VMEM((BPW, NUM_FIELDS), jnp.int32),
        pltpu.VMEM((BPW, CONT), jnp.float32),
        pltpu.VMEM((L,), jnp.float32),
        pltpu.VMEM((L,), jnp.float32),
        pltpu.VMEM((NROWS_IDX, 128), jnp.int32),
        pltpu.VMEM((NROWS_IDX, 128), jnp.float32),
        pltpu.VMEM((BPW,), jnp.float32),
        pltpu.SemaphoreType.DMA,
    ],
)
def _fm_sc(x_hbm, xc_hbm, tab_hbm, w_hbm, b_hbm, out_hbm,
           x_v, xc_v, w_v, b_v, idx_v, val_v, out_v, sem):
    _fm_body(x_hbm, xc_hbm, tab_hbm, w_hbm, b_hbm, out_hbm,
             x_v, xc_v, w_v, b_v, idx_v, val_v, out_v, sem)


def kernel(x, x_cont, linear_weight, bias, w):
    tab = linear_weight.reshape(-1)
    w16 = jnp.pad(w, (0, L - CONT))
    b16 = jnp.broadcast_to(bias, (L,))
    out = _fm_sc(x, x_cont, tab, w16, b16)
    return out.reshape(-1, 1)


# speedup vs baseline: 1.0062x; 1.0062x over previous
import functools

import jax
import jax.numpy as jnp
from jax import lax
from jax.experimental import pallas as pl
from jax.experimental.pallas import tpu as pltpu
from jax.experimental.pallas import tpu_sc as plsc

NUM_FIELDS = 26
FIELD_SIZE = 40000
CONT = 13
BATCH = 16384

NC = 2
NS = 16
L = 16
NW = NC * NS
BPW = BATCH // NW
NGB = BPW // 128
NROWS_IDX = NUM_FIELDS * NGB


def _fm_body(x_hbm, xc_hbm, tab_hbm, w_hbm, b_hbm, out_hbm,
             x_v, xc_v, w_v, b_v, idx_v, val_v, out_v, sem):
    cid = lax.axis_index("c")
    sid = lax.axis_index("s")
    wid = sid * NC + cid
    base = wid * BPW

    pltpu.sync_copy(x_hbm.at[pl.ds(base * NUM_FIELDS, BPW * NUM_FIELDS)], x_v)
    pltpu.sync_copy(xc_hbm.at[pl.ds(base * CONT, BPW * CONT)], xc_v)
    pltpu.sync_copy(w_hbm, w_v)
    pltpu.sync_copy(b_hbm, b_v)

    lanes = lax.iota(jnp.int32, 16)

    for g in range(NGB):
        def build(cc, carry, g=g):
            flat0 = ((g * 8 + cc) * L + lanes) * NUM_FIELDS
            lane0 = cc * L
            for j in range(NUM_FIELDS):
                xv = plsc.load_gather(x_v, [flat0 + j])
                idx_v[NGB * j + g, pl.ds(lane0, L)] = xv + (FIELD_SIZE * j)
            return carry
        lax.fori_loop(0, 8, build, 0)

    W = 16

    def fire(r, carry):
        pltpu.async_copy(tab_hbm.at[idx_v.at[r]], val_v.at[r], sem)
        return carry

    def drain_fire(r, carry):
        pltpu.make_async_copy(tab_hbm.at[idx_v.at[0]], val_v.at[0], sem).wait()
        pltpu.async_copy(tab_hbm.at[idx_v.at[r + W]], val_v.at[r + W], sem)
        return carry

    def drain(r, carry):
        pltpu.make_async_copy(tab_hbm.at[idx_v.at[0]], val_v.at[0], sem).wait()
        return carry

    lax.fori_loop(0, W, fire, 0)
    lax.fori_loop(0, NROWS_IDX - W, drain_fire, 0)
    lax.fori_loop(0, W, drain, 0)

    wjs = [w_v[pl.ds(j * L, L)] for j in range(CONT)]
    bias_vec = b_v[:]
    for g in range(NGB):
        def accum(cc, carry, g=g):
            lane0 = cc * L
            acc = bias_vec
            for j in range(NUM_FIELDS):
                acc = acc + val_v[NGB * j + g, pl.ds(lane0, L)]
            flat0 = ((g * 8 + cc) * L + lanes) * CONT
            for j in range(CONT):
                acc = acc + plsc.load_gather(xc_v, [flat0 + j]) * wjs[j]
            out_v[pl.ds(g * 128 + lane0, L)] = acc
            return carry
        lax.fori_loop(0, 8, accum, 0)

    pltpu.sync_copy(out_v, out_hbm.at[pl.ds(base, BPW)])


@functools.partial(
    pl.kernel,
    mesh=plsc.VectorSubcoreMesh(core_axis_name="c", subcore_axis_name="s"),
    out_type=jax.ShapeDtypeStruct((BATCH,), jnp.float32),
    compiler_params=pltpu.CompilerParams(needs_layout_passes=False),
    scratch_types=[
        pltpu.VMEM((BPW * NUM_FIELDS,), jnp.int32),
        pltpu.VMEM((BPW * CONT,), jnp.float32),
        pltpu.VMEM((CONT * L,), jnp.float32),
        pltpu.VMEM((L,), jnp.float32),
        pltpu.VMEM((NROWS_IDX, 128), jnp.int32),
        pltpu.VMEM((NROWS_IDX, 128), jnp.float32),
        pltpu.VMEM((BPW,), jnp.float32),
        pltpu.SemaphoreType.DMA,
    ],
)
def _fm_sc(x_hbm, xc_hbm, tab_hbm, w_hbm, b_hbm, out_hbm,
           x_v, xc_v, w_v, b_v, idx_v, val_v, out_v, sem):
    _fm_body(x_hbm, xc_hbm, tab_hbm, w_hbm, b_hbm, out_hbm,
             x_v, xc_v, w_v, b_v, idx_v, val_v, out_v, sem)


def kernel(x, x_cont, linear_weight, bias, w):
    tab = linear_weight.reshape(-1)
    w_t = jnp.broadcast_to(w.reshape(CONT, 1), (CONT, L)).reshape(-1)
    b16 = jnp.broadcast_to(bias, (L,))
    out = _fm_sc(x.reshape(-1), x_cont.reshape(-1), tab, w_t, b16)
    return out.reshape(-1, 1)


# speedup vs baseline: 1.1308x; 1.1239x over previous
import functools

import jax
import jax.numpy as jnp
from jax import lax
from jax.experimental import pallas as pl
from jax.experimental.pallas import tpu as pltpu
from jax.experimental.pallas import tpu_sc as plsc

NUM_FIELDS = 26
FIELD_SIZE = 40000
CONT = 13
BATCH = 16384
NUM_ROWS = NUM_FIELDS * FIELD_SIZE + 1

NC = 2
NS = 16
L = 16
NW = NC * NS
BPW = BATCH // NW
NGB = BPW // 128
NROWS_IDX = NUM_FIELDS * NGB


def _fm_body(x_hbm, xc_hbm, tab_hbm, w_hbm, b_hbm, out_hbm,
             x_v, xc_v, w_v, b_v, idx_v, val_v, out_v, sem):
    cid = lax.axis_index("c")
    sid = lax.axis_index("s")
    wid = sid * NC + cid
    base = wid * BPW

    pltpu.sync_copy(x_hbm.at[pl.ds(base, BPW), :], x_v)
    pltpu.sync_copy(xc_hbm.at[:, pl.ds(base, BPW)], xc_v)
    pltpu.sync_copy(w_hbm, w_v)
    pltpu.sync_copy(b_hbm, b_v)

    lanes = lax.iota(jnp.int32, 16)

    for g in range(NGB):
        def build(cc, carry, g=g):
            samp = (g * 8 + cc) * L + lanes
            lane0 = cc * L
            for j in range(NUM_FIELDS):
                col = jnp.full((L,), j, jnp.int32)
                xv = plsc.load_gather(x_v, [samp, col])
                idx_v[NGB * j + g, pl.ds(lane0, L)] = xv + (FIELD_SIZE * j)
            return carry
        lax.fori_loop(0, 8, build, 0)

    W = 16

    def fire(r, carry):
        pltpu.async_copy(tab_hbm.at[idx_v.at[r]], val_v.at[r], sem)
        return carry

    def drain_fire(r, carry):
        pltpu.make_async_copy(tab_hbm.at[idx_v.at[0]], val_v.at[0], sem).wait()
        pltpu.async_copy(tab_hbm.at[idx_v.at[r + W]], val_v.at[r + W], sem)
        return carry

    def drain(r, carry):
        pltpu.make_async_copy(tab_hbm.at[idx_v.at[0]], val_v.at[0], sem).wait()
        return carry

    lax.fori_loop(0, W, fire, 0)
    lax.fori_loop(0, NROWS_IDX - W, drain_fire, 0)
    lax.fori_loop(0, W, drain, 0)

    wjs = [w_v[pl.ds(j * L, L)] for j in range(CONT)]
    bias_vec = b_v[:]
    for g in range(NGB):
        def accum(cc, carry, g=g):
            lane0 = cc * L
            acc = bias_vec
            for j in range(NUM_FIELDS):
                acc = acc + val_v[NGB * j + g, pl.ds(lane0, L)]
            for j in range(CONT):
                acc = acc + xc_v[j, pl.ds(g * 128 + lane0, L)] * wjs[j]
            out_v[pl.ds(g * 128 + lane0, L)] = acc
            return carry
        lax.fori_loop(0, 8, accum, 0)

    pltpu.sync_copy(out_v, out_hbm.at[pl.ds(base, BPW)])


@functools.partial(
    pl.kernel,
    mesh=plsc.VectorSubcoreMesh(core_axis_name="c", subcore_axis_name="s"),
    out_type=jax.ShapeDtypeStruct((BATCH,), jnp.float32),
    compiler_params=pltpu.CompilerParams(needs_layout_passes=False),
    scratch_types=[
        pltpu.VMEM((BPW, NUM_FIELDS), jnp.int32),
        pltpu.VMEM((CONT, BPW), jnp.float32),
        pltpu.VMEM((CONT * L,), jnp.float32),
        pltpu.VMEM((L,), jnp.float32),
        pltpu.VMEM((NROWS_IDX, 128), jnp.int32),
        pltpu.VMEM((NROWS_IDX, 128), jnp.float32),
        pltpu.VMEM((BPW,), jnp.float32),
        pltpu.SemaphoreType.DMA,
    ],
)
def _fm_sc(x_hbm, xc_hbm, tab_hbm, w_hbm, b_hbm, out_hbm,
           x_v, xc_v, w_v, b_v, idx_v, val_v, out_v, sem):
    _fm_body(x_hbm, xc_hbm, tab_hbm, w_hbm, b_hbm, out_hbm,
             x_v, xc_v, w_v, b_v, idx_v, val_v, out_v, sem)


def kernel(x, x_cont, linear_weight, bias, w):
    w_t = jnp.broadcast_to(w.reshape(CONT, 1), (CONT, L)).reshape(-1)
    b16 = jnp.broadcast_to(bias, (L,))
    out = _fm_sc(x, x_cont.T, linear_weight.reshape(-1), w_t, b16)
    return out.reshape(-1, 1)


# speedup vs baseline: 2.5625x; 2.2661x over previous
import functools

import jax
import jax.numpy as jnp
from jax import lax
from jax.experimental import pallas as pl
from jax.experimental.pallas import tpu as pltpu
from jax.experimental.pallas import tpu_sc as plsc

NUM_FIELDS = 26
FIELD_SIZE = 40000
CONT = 13
BATCH = 16384
NUM_ROWS = NUM_FIELDS * FIELD_SIZE + 1

NC = 2
NS = 16
L = 16
NW = NC * NS
BPW = BATCH // NW
NGB = BPW // 128
NROWS_IDX = NUM_FIELDS * NGB


def _fm_body(x_hbm, xc_hbm, tab_hbm, w_hbm, b_hbm, out_hbm,
             x_v, xc_v, w_v, b_v, idx_v, val_v, out_v, sem):
    cid = lax.axis_index("c")
    sid = lax.axis_index("s")
    wid = sid * NC + cid
    base = wid * BPW

    pltpu.sync_copy(x_hbm.at[:, pl.ds(base, BPW)], x_v)
    pltpu.sync_copy(xc_hbm.at[:, pl.ds(base, BPW)], xc_v)
    pltpu.sync_copy(w_hbm, w_v)
    pltpu.sync_copy(b_hbm, b_v)

    lanes = lax.iota(jnp.int32, 16)
    tab1 = tab_hbm.at[0]

    for g in range(NGB):
        def build(cc, carry, g=g):
            pos = g * 128 + cc * L
            lane0 = cc * L
            for j in range(NUM_FIELDS):
                xv = x_v[j, pl.ds(pos, L)]
                idx_v[NGB * j + g, pl.ds(lane0, L)] = xv + (FIELD_SIZE * j)
            return carry
        lax.fori_loop(0, 8, build, 0)

    W = 16

    def fire(r, carry):
        pltpu.async_copy(tab1.at[idx_v.at[r]], val_v.at[r], sem)
        return carry

    def drain_fire(r, carry):
        pltpu.make_async_copy(tab1.at[idx_v.at[0]], val_v.at[0], sem).wait()
        pltpu.async_copy(tab1.at[idx_v.at[r + W]], val_v.at[r + W], sem)
        return carry

    def drain(r, carry):
        pltpu.make_async_copy(tab1.at[idx_v.at[0]], val_v.at[0], sem).wait()
        return carry

    lax.fori_loop(0, W, fire, 0)
    lax.fori_loop(0, NROWS_IDX - W, drain_fire, 0)
    lax.fori_loop(0, W, drain, 0)

    wjs = [w_v[pl.ds(j * L, L)] for j in range(CONT)]
    bias_vec = b_v[:]
    for g in range(NGB):
        def accum(cc, carry, g=g):
            lane0 = cc * L
            acc = bias_vec
            for j in range(NUM_FIELDS):
                acc = acc + val_v[NGB * j + g, pl.ds(lane0, L)]
            for j in range(CONT):
                acc = acc + xc_v[j, pl.ds(g * 128 + lane0, L)] * wjs[j]
            out_v[pl.ds(g * 128 + lane0, L)] = acc
            return carry
        lax.fori_loop(0, 8, accum, 0)

    pltpu.sync_copy(out_v, out_hbm.at[pl.ds(base, BPW)])


@functools.partial(
    pl.kernel,
    mesh=plsc.VectorSubcoreMesh(core_axis_name="c", subcore_axis_name="s"),
    out_type=jax.ShapeDtypeStruct((BATCH,), jnp.float32),
    compiler_params=pltpu.CompilerParams(needs_layout_passes=False),
    scratch_types=[
        pltpu.VMEM((NUM_FIELDS, BPW), jnp.int32),
        pltpu.VMEM((CONT, BPW), jnp.float32),
        pltpu.VMEM((CONT * L,), jnp.float32),
        pltpu.VMEM((L,), jnp.float32),
        pltpu.VMEM((NROWS_IDX, 128), jnp.int32),
        pltpu.VMEM((NROWS_IDX, 128), jnp.float32),
        pltpu.VMEM((BPW,), jnp.float32),
        pltpu.SemaphoreType.DMA,
    ],
)
def _fm_sc(x_hbm, xc_hbm, tab_hbm, w_hbm, b_hbm, out_hbm,
           x_v, xc_v, w_v, b_v, idx_v, val_v, out_v, sem):
    _fm_body(x_hbm, xc_hbm, tab_hbm, w_hbm, b_hbm, out_hbm,
             x_v, xc_v, w_v, b_v, idx_v, val_v, out_v, sem)


def kernel(x, x_cont, linear_weight, bias, w):
    w_t = jnp.broadcast_to(w.reshape(CONT, 1), (CONT, L)).reshape(-1)
    b16 = jnp.broadcast_to(bias, (L,))
    out = _fm_sc(x.T, x_cont.T, linear_weight.T, w_t, b16)
    return out.reshape(-1, 1)


# speedup vs baseline: 2.9683x; 1.1584x over previous
import functools

import jax
import jax.numpy as jnp
from jax import lax
from jax.experimental import pallas as pl
from jax.experimental.pallas import tpu as pltpu
from jax.experimental.pallas import tpu_sc as plsc

NUM_FIELDS = 26
FIELD_SIZE = 40000
CONT = 13
BATCH = 16384
NUM_ROWS = NUM_FIELDS * FIELD_SIZE + 1

NC = 2
NS = 16
L = 16
NW = NC * NS
BPW = BATCH // NW
NGB = BPW // 128
NROWS_IDX = NUM_FIELDS * NGB


def _fm_body(x_hbm, xc_hbm, tab_hbm, w_hbm, b_hbm, out_hbm,
             x_v, xc_v, w_v, b_v, val_v, out_v, sem):
    cid = lax.axis_index("c")
    sid = lax.axis_index("s")
    wid = sid * NC + cid
    base = wid * BPW

    pltpu.sync_copy(x_hbm.at[:, pl.ds(base, BPW)], x_v)
    pltpu.sync_copy(xc_hbm.at[:, pl.ds(base, BPW)], xc_v)
    pltpu.sync_copy(w_hbm, w_v)
    pltpu.sync_copy(b_hbm, b_v)

    tab1 = tab_hbm.at[0]

    def row_copy(j, g, sem):
        src = tab1.at[pl.ds(FIELD_SIZE * j, FIELD_SIZE)]
        return pltpu.make_async_copy(
            src.at[x_v.at[j, pl.ds(g * 128, 128)]],
            val_v.at[NGB * j + g], sem)

    for j in range(NUM_FIELDS):
        for g in range(NGB):
            row_copy(j, g, sem).start()
    for j in range(NUM_FIELDS):
        for g in range(NGB):
            row_copy(j, g, sem).wait()

    wjs = [w_v[pl.ds(j * L, L)] for j in range(CONT)]
    bias_vec = b_v[:]
    for g in range(NGB):
        def accum(cc, carry, g=g):
            lane0 = cc * L
            acc = bias_vec
            for j in range(NUM_FIELDS):
                acc = acc + val_v[NGB * j + g, pl.ds(lane0, L)]
            for j in range(CONT):
                acc = acc + xc_v[j, pl.ds(g * 128 + lane0, L)] * wjs[j]
            out_v[pl.ds(g * 128 + lane0, L)] = acc
            return carry
        lax.fori_loop(0, 8, accum, 0)

    pltpu.sync_copy(out_v, out_hbm.at[pl.ds(base, BPW)])


@functools.partial(
    pl.kernel,
    mesh=plsc.VectorSubcoreMesh(core_axis_name="c", subcore_axis_name="s"),
    out_type=jax.ShapeDtypeStruct((BATCH,), jnp.float32),
    compiler_params=pltpu.CompilerParams(needs_layout_passes=False),
    scratch_types=[
        pltpu.VMEM((NUM_FIELDS, BPW), jnp.int32),
        pltpu.VMEM((CONT, BPW), jnp.float32),
        pltpu.VMEM((CONT * L,), jnp.float32),
        pltpu.VMEM((L,), jnp.float32),
        pltpu.VMEM((NROWS_IDX, 128), jnp.float32),
        pltpu.VMEM((BPW,), jnp.float32),
        pltpu.SemaphoreType.DMA,
    ],
)
def _fm_sc(x_hbm, xc_hbm, tab_hbm, w_hbm, b_hbm, out_hbm,
           x_v, xc_v, w_v, b_v, val_v, out_v, sem):
    _fm_body(x_hbm, xc_hbm, tab_hbm, w_hbm, b_hbm, out_hbm,
             x_v, xc_v, w_v, b_v, val_v, out_v, sem)


def kernel(x, x_cont, linear_weight, bias, w):
    w_t = jnp.broadcast_to(w.reshape(CONT, 1), (CONT, L)).reshape(-1)
    b16 = jnp.broadcast_to(bias, (L,))
    out = _fm_sc(x.T, x_cont.T, linear_weight.T, w_t, b16)
    return out.reshape(-1, 1)


# speedup vs baseline: 3.0795x; 1.0375x over previous
import functools

import jax
import jax.numpy as jnp
from jax import lax
from jax.experimental import pallas as pl
from jax.experimental.pallas import tpu as pltpu
from jax.experimental.pallas import tpu_sc as plsc

NUM_FIELDS = 26
FIELD_SIZE = 40000
CONT = 13
BATCH = 16384
NUM_ROWS = NUM_FIELDS * FIELD_SIZE + 1

NC = 2
NS = 16
L = 16
NW = NC * NS
BPW = BATCH // NW
NGB = BPW // 128
NROWS_IDX = NUM_FIELDS * NGB


def _fm_body(x_hbm, xc_hbm, tab_hbm, wb_hbm, out_hbm,
             x_v, xc_v, wb_v, val_v, out_v, sem):
    cid = lax.axis_index("c")
    sid = lax.axis_index("s")
    wid = sid * NC + cid
    base = wid * BPW

    pltpu.sync_copy(x_hbm.at[:, pl.ds(base, BPW)], x_v)
    pltpu.sync_copy(xc_hbm.at[:, pl.ds(base, BPW)], xc_v)
    pltpu.sync_copy(wb_hbm, wb_v)

    tab1 = tab_hbm.at[0]

    def row_copy(j, g, sem):
        off = pl.multiple_of(j * FIELD_SIZE, 8)
        src = tab1.at[pl.ds(off, FIELD_SIZE)]
        return pltpu.make_async_copy(
            src.at[x_v.at[j, pl.ds(g * 128, 128)]],
            val_v.at[j, pl.ds(g * 128, 128)], sem)

    def fire(r, carry):
        row_copy(lax.shift_right_logical(r, 2), r & 3, sem).start()
        return carry

    def drain(r, carry):
        row_copy(0, 0, sem).wait()
        return carry

    lax.fori_loop(0, NROWS_IDX, fire, 0)
    lax.fori_loop(0, NROWS_IDX, drain, 0)

    wbv = wb_v[:]
    ws = [wbv[j] for j in range(CONT)]
    b0 = wbv[CONT]
    for g in range(NGB):
        def accum(cc, carry, g=g):
            pos = g * 128 + cc * L
            acc = jnp.full((L,), b0, jnp.float32)
            for j in range(NUM_FIELDS):
                acc = acc + val_v[j, pl.ds(pos, L)]
            for j in range(CONT):
                acc = acc + xc_v[j, pl.ds(pos, L)] * ws[j]
            out_v[pl.ds(pos, L)] = acc
            return carry
        lax.fori_loop(0, 8, accum, 0)

    pltpu.sync_copy(out_v, out_hbm.at[pl.ds(base, BPW)])


@functools.partial(
    pl.kernel,
    mesh=plsc.VectorSubcoreMesh(core_axis_name="c", subcore_axis_name="s"),
    out_type=jax.ShapeDtypeStruct((BATCH,), jnp.float32),
    compiler_params=pltpu.CompilerParams(needs_layout_passes=False),
    scratch_types=[
        pltpu.VMEM((NUM_FIELDS, BPW), jnp.int32),
        pltpu.VMEM((CONT, BPW), jnp.float32),
        pltpu.VMEM((L,), jnp.float32),
        pltpu.VMEM((NUM_FIELDS, BPW), jnp.float32),
        pltpu.VMEM((BPW,), jnp.float32),
        pltpu.SemaphoreType.DMA,
    ],
)
def _fm_sc(x_hbm, xc_hbm, tab_hbm, wb_hbm, out_hbm,
           x_v, xc_v, wb_v, val_v, out_v, sem):
    _fm_body(x_hbm, xc_hbm, tab_hbm, wb_hbm, out_hbm,
             x_v, xc_v, wb_v, val_v, out_v, sem)


def kernel(x, x_cont, linear_weight, bias, w):
    wb = jnp.concatenate([w, bias, jnp.zeros((L - CONT - 1,), jnp.float32)])
    out = _fm_sc(x.T, x_cont.T, linear_weight.T, wb)
    return out.reshape(-1, 1)


# speedup vs baseline: 3.1203x; 1.0132x over previous
import functools

import jax
import jax.numpy as jnp
from jax import lax
from jax.experimental import pallas as pl
from jax.experimental.pallas import tpu as pltpu
from jax.experimental.pallas import tpu_sc as plsc

NUM_FIELDS = 26
FIELD_SIZE = 40000
CONT = 13
BATCH = 16384
NUM_ROWS = NUM_FIELDS * FIELD_SIZE + 1

NC = 2
NS = 16
L = 16
NW = NC * NS
BPW = BATCH // NW
NGB = BPW // 128
NROWS_IDX = NUM_FIELDS * NGB


def _fm_body(x_hbm, xc_hbm, tab_hbm, wb_hbm, out_hbm,
             x_v, xc_v, wb_v, val_v, out_v, sem):
    cid = lax.axis_index("c")
    sid = lax.axis_index("s")
    wid = sid * NC + cid
    base = wid * BPW

    pltpu.sync_copy(x_hbm.at[:, pl.ds(base, BPW)], x_v)
    pltpu.sync_copy(xc_hbm.at[:, pl.ds(base, BPW)], xc_v)
    pltpu.sync_copy(wb_hbm, wb_v)

    tab1 = tab_hbm.at[0]

    def row_copy(j, g, sem):
        off = pl.multiple_of(j * FIELD_SIZE, 8)
        src = tab1.at[pl.ds(off, FIELD_SIZE)]
        return pltpu.make_async_copy(
            src.at[x_v.at[j, pl.ds(g * 128, 128)]],
            val_v.at[j, pl.ds(g * 128, 128)], sem)

    def fire(r, carry):
        row_copy(lax.shift_right_logical(r, 2), r & 3, sem).start()
        return carry

    def drain(r, carry):
        row_copy(0, 0, sem).wait()
        return carry

    lax.fori_loop(0, NROWS_IDX, fire, 0)
    lax.fori_loop(0, NROWS_IDX, drain, 0)

    wbv = wb_v[:]
    ws = [wbv[j] for j in range(CONT)]
    b0 = wbv[CONT]

    def accum(c, carry):
        pos = c * L
        acc = jnp.full((L,), b0, jnp.float32)
        for j in range(NUM_FIELDS):
            acc = acc + val_v[j, pl.ds(pos, L)]
        for j in range(CONT):
            acc = acc + xc_v[j, pl.ds(pos, L)] * ws[j]
        out_v[pl.ds(pos, L)] = acc
        return carry
    lax.fori_loop(0, BPW // L, accum, 0)

    pltpu.sync_copy(out_v, out_hbm.at[pl.ds(base, BPW)])


@functools.partial(
    pl.kernel,
    mesh=plsc.VectorSubcoreMesh(core_axis_name="c", subcore_axis_name="s"),
    out_type=jax.ShapeDtypeStruct((BATCH,), jnp.float32),
    compiler_params=pltpu.CompilerParams(needs_layout_passes=False),
    scratch_types=[
        pltpu.VMEM((NUM_FIELDS, BPW), jnp.int32),
        pltpu.VMEM((CONT, BPW), jnp.float32),
        pltpu.VMEM((L,), jnp.float32),
        pltpu.VMEM((NUM_FIELDS, BPW), jnp.float32),
        pltpu.VMEM((BPW,), jnp.float32),
        pltpu.SemaphoreType.DMA,
    ],
)
def _fm_sc(x_hbm, xc_hbm, tab_hbm, wb_hbm, out_hbm,
           x_v, xc_v, wb_v, val_v, out_v, sem):
    _fm_body(x_hbm, xc_hbm, tab_hbm, wb_hbm, out_hbm,
             x_v, xc_v, wb_v, val_v, out_v, sem)


def kernel(x, x_cont, linear_weight, bias, w):
    wb = jnp.concatenate([w, bias, jnp.zeros((L - CONT - 1,), jnp.float32)])
    out = _fm_sc(x.T, x_cont.T, linear_weight.T, wb)
    return out.reshape(-1, 1)


# speedup vs baseline: 3.1239x; 1.0011x over previous
import functools

import jax
import jax.numpy as jnp
from jax import lax
from jax.experimental import pallas as pl
from jax.experimental.pallas import tpu as pltpu
from jax.experimental.pallas import tpu_sc as plsc

NUM_FIELDS = 26
FIELD_SIZE = 40000
CONT = 13
BATCH = 16384
NUM_ROWS = NUM_FIELDS * FIELD_SIZE + 1

NC = 2
NS = 16
L = 16
NW = NC * NS
BPW = BATCH // NW
NGB = BPW // 128
NROWS_IDX = NUM_FIELDS * NGB


def _fm_body(x_hbm, xc_hbm, tab_hbm, w_hbm, b_hbm, out_hbm,
             x_v, xc_v, w_v, b_v, val_v, out_v, sem):
    cid = lax.axis_index("c")
    sid = lax.axis_index("s")
    wid = sid * NC + cid
    base = wid * BPW

    pltpu.sync_copy(x_hbm.at[:, pl.ds(base, BPW)], x_v)
    pltpu.sync_copy(xc_hbm.at[:, pl.ds(base, BPW)], xc_v)
    pltpu.sync_copy(w_hbm, w_v.at[pl.ds(0, CONT)])
    pltpu.sync_copy(b_hbm, b_v.at[pl.ds(0, 1)])

    tab1 = tab_hbm.at[0]

    def row_copy(j, g, sem):
        off = pl.multiple_of(j * FIELD_SIZE, 8)
        src = tab1.at[pl.ds(off, FIELD_SIZE)]
        return pltpu.make_async_copy(
            src.at[x_v.at[j, pl.ds(g * 128, 128)]],
            val_v.at[j, pl.ds(g * 128, 128)], sem)

    def fire(r, carry):
        row_copy(lax.shift_right_logical(r, 2), r & 3, sem).start()
        return carry

    def drain(r, carry):
        row_copy(0, 0, sem).wait()
        return carry

    lax.fori_loop(0, NROWS_IDX, fire, 0)
    lax.fori_loop(0, NROWS_IDX, drain, 0)

    wv = w_v[:]
    ws = [wv[j] for j in range(CONT)]
    b0 = b_v[:][0]

    def accum(c, carry):
        pos = c * L
        acc = jnp.full((L,), b0, jnp.float32)
        for j in range(NUM_FIELDS):
            acc = acc + val_v[j, pl.ds(pos, L)]
        for j in range(CONT):
            acc = acc + xc_v[j, pl.ds(pos, L)] * ws[j]
        out_v[pl.ds(pos, L)] = acc
        return carry
    lax.fori_loop(0, BPW // L, accum, 0)

    pltpu.sync_copy(out_v, out_hbm.at[pl.ds(base, BPW)])


@functools.partial(
    pl.kernel,
    mesh=plsc.VectorSubcoreMesh(core_axis_name="c", subcore_axis_name="s"),
    out_type=jax.ShapeDtypeStruct((BATCH,), jnp.float32),
    compiler_params=pltpu.CompilerParams(needs_layout_passes=False),
    scratch_types=[
        pltpu.VMEM((NUM_FIELDS, BPW), jnp.int32),
        pltpu.VMEM((CONT, BPW), jnp.float32),
        pltpu.VMEM((L,), jnp.float32),
        pltpu.VMEM((L,), jnp.float32),
        pltpu.VMEM((NUM_FIELDS, BPW), jnp.float32),
        pltpu.VMEM((BPW,), jnp.float32),
        pltpu.SemaphoreType.DMA,
    ],
)
def _fm_sc(x_hbm, xc_hbm, tab_hbm, w_hbm, b_hbm, out_hbm,
           x_v, xc_v, w_v, b_v, val_v, out_v, sem):
    _fm_body(x_hbm, xc_hbm, tab_hbm, w_hbm, b_hbm, out_hbm,
             x_v, xc_v, w_v, b_v, val_v, out_v, sem)


def kernel(x, x_cont, linear_weight, bias, w):
    out = _fm_sc(x.T, x_cont.T, linear_weight.T, w, bias)
    return out.reshape(-1, 1)
